# baseline (speedup 1.0000x reference)
import functools

import jax
import jax.numpy as jnp
from jax import lax
from jax.experimental import pallas as pl
from jax.experimental.pallas import tpu as pltpu
from jax.experimental.pallas import tpu_sc as plsc

R, N = 128, 32768
L = 16
NV = N // L
SEG = 512
REG_K = 16
UNROLL = 8
NB = NV // UNROLL
N_BISECT = 18
NEG = -1.0e30

_mesh = plsc.VectorSubcoreMesh(core_axis_name="c", subcore_axis_name="s")


def _all_reduce(a, op):
    idx0 = lax.iota(jnp.int32, L)
    for k in (8, 4, 2, 1):
        perm = jnp.bitwise_xor(idx0, k)
        a = op(a, jnp.take_along_axis(a, perm, axis=0))
    return a


def _tree_sum(xs):
    xs = list(xs)
    while len(xs) > 1:
        xs = [xs[i] + xs[i + 1] for i in range(0, len(xs) - 1, 2)] + (
            [xs[-1]] if len(xs) % 2 else []
        )
    return xs[0]


def _bisect(lo, hi, eval_g):
    def body(_, lohi):
        lo, hi = lohi
        tau = 0.5 * (lo + hi)
        big = eval_g(tau)
        return jnp.where(big, tau, lo), jnp.where(big, hi, tau)

    lo, hi = lax.fori_loop(0, N_BISECT, body, (lo, hi))
    return 0.5 * (lo + hi)


@functools.partial(
    pl.kernel,
    mesh=_mesh,
    out_type=jax.ShapeDtypeStruct((R, N), jnp.float32),
    scratch_types=[
        pltpu.VMEM((N,), jnp.float32),
        pltpu.VMEM((N,), jnp.float32),
        pltpu.VMEM((N,), jnp.float32),
        pltpu.VMEM((SEG * L + L,), jnp.float32),
        pltpu.VMEM((NB * L + L,), jnp.int32),
        pltpu.SemaphoreType.DMA,
        pltpu.SemaphoreType.DMA,
        pltpu.SemaphoreType.DMA,
        pltpu.SemaphoreType.DMA,
        pltpu.SemaphoreType.DMA,
        pltpu.SemaphoreType.DMA,
    ],
    compiler_params=pltpu.CompilerParams(needs_layout_passes=False),
)
def _sparsemax_sc(x_hbm, out_hbm, row_a, row_b, row_c, cmp_v, blk_v,
                  si0, si1, si2, so0, so1, so2):
    info = plsc.get_sparse_core_info()
    nc, ns = info.num_cores, info.num_subcores
    nw = nc * ns
    rows_per = R // nw
    wid = lax.axis_index("s") * nc + lax.axis_index("c")
    r0 = wid * rows_per
    lanes = lax.iota(jnp.int32, L)
    dump = SEG * L + lanes
    dump_b = NB * L + lanes

    def compute_row(buf):
        sm0 = tuple(buf[pl.ds(u * UNROLL * L, L)] for u in range(4))

        @plsc.parallel_loop(1, NB // 4, carry=sm0, unroll=2)
        def sm(i, sm):
            base = i * (4 * UNROLL * L)
            return tuple(
                jnp.maximum(sm[u], buf[pl.ds(base + u * UNROLL * L, L)])
                for u in range(4)
            )

        smax = jnp.maximum(jnp.maximum(sm[0], sm[1]),
                           jnp.maximum(sm[2], sm[3]))
        t = _all_reduce(smax, jnp.maximum) - 1.0

        @plsc.parallel_loop(0, NB, carry=jnp.zeros((L,), jnp.int32), unroll=2)
        def boff(i, boff):
            base = i * (UNROLL * L)
            hots = [buf[pl.ds(base + u * L, L)] > t for u in range(UNROLL)]
            while len(hots) > 1:
                hots = [hots[k] | hots[k + 1] for k in range(0, len(hots), 2)]
            anyh = hots[0]
            bidx = jnp.where(anyh, boff * L + lanes, dump_b)
            plsc.store_scatter(blk_v, [bidx], jnp.zeros((L,), jnp.int32) + i)
            return boff + anyh.astype(jnp.int32)

        max_boff = _all_reduce(boff, jnp.maximum)[0]

        carry0 = (jnp.zeros((L,), jnp.int32),
                  jnp.full((L,), -jnp.inf, jnp.float32))

        @plsc.parallel_loop(0, max_boff, carry=carry0)
        def off_gm(k, c):
            off, gm = c
            bid = plsc.load_gather(blk_v, [k * L + lanes])
            bid = jnp.minimum(jnp.maximum(bid, 0), NB - 1)
            valid = k < boff
            for u in range(UNROLL):
                v = plsc.load_gather(buf, [(bid * UNROLL + u) * L + lanes])
                gm = jnp.maximum(gm, jnp.where(valid, v, -jnp.inf))
                hot = (v > t) & valid
                slot = jnp.minimum(off, SEG - 1)
                idx = jnp.where(hot, slot * L + lanes, dump)
                plsc.store_scatter(cmp_v, [idx], v)
                off = off + hot.astype(jnp.int32)
            return off, gm

        off, gm = off_gm
        mx = _all_reduce(gm, jnp.maximum)
        max_off = _all_reduce(off, jnp.maximum)[0]

        vals = tuple(
            jnp.where(kk < off, cmp_v[pl.ds(kk * L, L)], NEG)
            for kk in range(REG_K)
        )

        def eval_g_reg(tau):
            accs = [jnp.zeros((L,), jnp.float32) for _ in range(4)]
            for kk in range(REG_K):
                accs[kk % 4] = accs[kk % 4] + jnp.maximum(vals[kk] - tau, 0.0)
            return _all_reduce(_tree_sum(accs), jnp.add) > 1.0

        def eval_g_loop(tau):
            @plsc.parallel_loop(0, max_off, carry=jnp.zeros((L,), jnp.float32))
            def a(kk, a):
                v = jnp.where(kk < off, cmp_v[pl.ds(kk * L, L)], NEG)
                return a + jnp.maximum(v - tau, 0.0)

            return _all_reduce(a, jnp.add) > 1.0

        def eval_g_full(tau):
            acc0 = tuple(jnp.zeros((L,), jnp.float32) for _ in range(UNROLL))

            @plsc.parallel_loop(0, NV // UNROLL, carry=acc0, unroll=2)
            def accs(i, accs):
                base = i * (UNROLL * L)
                return tuple(
                    accs[u]
                    + jnp.maximum(buf[pl.ds(base + u * L, L)] - tau, 0.0)
                    for u in range(UNROLL)
                )

            a = list(accs)
            step = UNROLL
            while step > 1:
                step //= 2
                a = [a[u] + a[u + step] for u in range(step)]
            return _all_reduce(a[0], jnp.add) > 1.0

        tau = lax.cond(
            max_off <= REG_K,
            lambda: _bisect(mx - 1.0, mx, eval_g_reg),
            lambda: lax.cond(
                max_off <= SEG,
                lambda: _bisect(mx - 1.0, mx, eval_g_loop),
                lambda: _bisect(mx - 1.0, mx, eval_g_full),
            ),
        )

        @plsc.parallel_loop(0, NV // UNROLL, unroll=2)
        def _(i):
            base = i * (UNROLL * L)
            for u in range(UNROLL):
                sl = pl.ds(base + u * L, L)
                buf[sl] = jnp.maximum(buf[sl] - tau, 0.0)

    bufs = (row_a, row_b, row_c)
    in_sems = (si0, si1, si2)
    out_sems = (so0, so1, so2)
    in_cp = [None] * rows_per
    out_cp = [None] * rows_per
    in_cp[0] = pltpu.async_copy(x_hbm.at[r0], bufs[0], in_sems[0])
    in_cp[1] = pltpu.async_copy(x_hbm.at[r0 + 1], bufs[1], in_sems[1])
    for j in range(rows_per):
        buf = bufs[j % 3]
        in_cp[j].wait()
        compute_row(buf)
        if j + 2 < rows_per:
            if j >= 1:
                out_cp[j - 1].wait()
            in_cp[j + 2] = pltpu.async_copy(
                x_hbm.at[r0 + j + 2], bufs[(j + 2) % 3], in_sems[(j + 2) % 3]
            )
        out_cp[j] = pltpu.async_copy(buf, out_hbm.at[r0 + j], out_sems[j % 3])
    for j in range(max(1, rows_per - 3), rows_per):
        out_cp[j].wait()


def kernel(input):
    return _sparsemax_sc(input)


# speedup vs baseline: 1.1922x; 1.1922x over previous
import functools

import jax
import jax.numpy as jnp
from jax import lax
from jax.experimental import pallas as pl
from jax.experimental.pallas import tpu as pltpu
from jax.experimental.pallas import tpu_sc as plsc

R, N = 128, 32768
L = 16
NV = N // L
SEG = 512
REG_K = 16
UNROLL = 8
NB = NV // UNROLL
N_BISECT = 18
NEG = -1.0e30

_mesh = plsc.VectorSubcoreMesh(core_axis_name="c", subcore_axis_name="s")


def _all_reduce(a, op):
    idx0 = lax.iota(jnp.int32, L)
    for k in (8, 4, 2, 1):
        perm = jnp.bitwise_xor(idx0, k)
        a = op(a, jnp.take_along_axis(a, perm, axis=0))
    return a


def _tree_sum(xs):
    xs = list(xs)
    while len(xs) > 1:
        xs = [xs[i] + xs[i + 1] for i in range(0, len(xs) - 1, 2)] + (
            [xs[-1]] if len(xs) % 2 else []
        )
    return xs[0]


def _bisect(lo, hi, eval_g):
    def body(_, lohi):
        lo, hi = lohi
        tau = 0.5 * (lo + hi)
        big = eval_g(tau)
        return jnp.where(big, tau, lo), jnp.where(big, hi, tau)

    lo, hi = lax.fori_loop(0, N_BISECT, body, (lo, hi))
    return 0.5 * (lo + hi)


@functools.partial(
    pl.kernel,
    mesh=_mesh,
    out_type=jax.ShapeDtypeStruct((R, N), jnp.float32),
    scratch_types=[
        pltpu.VMEM((N,), jnp.float32),
        pltpu.VMEM((N,), jnp.float32),
        pltpu.VMEM((N,), jnp.float32),
        pltpu.VMEM((SEG * L + L,), jnp.float32),
        pltpu.VMEM((NB * L + L,), jnp.int32),
    ] + [pltpu.SemaphoreType.DMA] * 12,
    compiler_params=pltpu.CompilerParams(needs_layout_passes=False),
)
def _sparsemax_sc(x_hbm, out_hbm, row_a, row_b, row_c, cmp_v, blk_v, *sems):
    info = plsc.get_sparse_core_info()
    nc, ns = info.num_cores, info.num_subcores
    nw = nc * ns
    rows_per = R // nw
    wid = lax.axis_index("s") * nc + lax.axis_index("c")
    r0 = wid * rows_per
    lanes = lax.iota(jnp.int32, L)
    dump = SEG * L + lanes
    dump_b = NB * L + lanes

    def compute_row(buf, mid_wait, out_issue):
        ms0 = tuple(buf[pl.ds(u * L, L)] for u in range(UNROLL))

        @plsc.parallel_loop(1, NB // 2, carry=ms0, unroll=2)
        def ms_half(i, ms):
            base = i * (UNROLL * L)
            return tuple(
                jnp.maximum(ms[u], buf[pl.ds(base + u * L, L)])
                for u in range(UNROLL)
            )

        mid_wait()

        @plsc.parallel_loop(NB // 2, NB, carry=ms_half, unroll=2)
        def ms(i, ms):
            base = i * (UNROLL * L)
            return tuple(
                jnp.maximum(ms[u], buf[pl.ds(base + u * L, L)])
                for u in range(UNROLL)
            )

        step = UNROLL
        while step > 1:
            step //= 2
            ms = tuple(jnp.maximum(ms[u], ms[u + step]) for u in range(step))
        mx = _all_reduce(ms[0], jnp.maximum)

        thr = mx - 1.0

        @plsc.parallel_loop(0, NB, carry=jnp.zeros((L,), jnp.int32), unroll=2)
        def boff(i, boff):
            base = i * (UNROLL * L)
            hots = [buf[pl.ds(base + u * L, L)] > thr for u in range(UNROLL)]
            while len(hots) > 1:
                hots = [hots[k] | hots[k + 1] for k in range(0, len(hots), 2)]
            anyh = hots[0]
            bidx = jnp.where(anyh, boff * L + lanes, dump_b)
            plsc.store_scatter(blk_v, [bidx], jnp.zeros((L,), jnp.int32) + i)
            return boff + anyh.astype(jnp.int32)

        max_boff = _all_reduce(boff, jnp.maximum)[0]

        @plsc.parallel_loop(0, max_boff, carry=jnp.zeros((L,), jnp.int32))
        def off(k, off):
            bid = plsc.load_gather(blk_v, [k * L + lanes])
            bid = jnp.minimum(jnp.maximum(bid, 0), NB - 1)
            valid = k < boff
            for u in range(UNROLL):
                v = plsc.load_gather(buf, [(bid * UNROLL + u) * L + lanes])
                hot = (v > thr) & valid
                slot = jnp.minimum(off, SEG - 1)
                idx = jnp.where(hot, slot * L + lanes, dump)
                plsc.store_scatter(cmp_v, [idx], v)
                off = off + hot.astype(jnp.int32)
            return off

        max_off = _all_reduce(off, jnp.maximum)[0]

        vals = tuple(
            jnp.where(kk < off, cmp_v[pl.ds(kk * L, L)], NEG)
            for kk in range(REG_K)
        )

        def eval_g_reg(tau):
            accs = [jnp.zeros((L,), jnp.float32) for _ in range(4)]
            for kk in range(REG_K):
                accs[kk % 4] = accs[kk % 4] + jnp.maximum(vals[kk] - tau, 0.0)
            return _all_reduce(_tree_sum(accs), jnp.add) > 1.0

        def eval_g_loop(tau):
            @plsc.parallel_loop(0, max_off, carry=jnp.zeros((L,), jnp.float32))
            def a(kk, a):
                v = jnp.where(kk < off, cmp_v[pl.ds(kk * L, L)], NEG)
                return a + jnp.maximum(v - tau, 0.0)

            return _all_reduce(a, jnp.add) > 1.0

        def eval_g_full(tau):
            acc0 = tuple(jnp.zeros((L,), jnp.float32) for _ in range(UNROLL))

            @plsc.parallel_loop(0, NV // UNROLL, carry=acc0, unroll=2)
            def accs(i, accs):
                base = i * (UNROLL * L)
                return tuple(
                    accs[u]
                    + jnp.maximum(buf[pl.ds(base + u * L, L)] - tau, 0.0)
                    for u in range(UNROLL)
                )

            a = list(accs)
            step = UNROLL
            while step > 1:
                step //= 2
                a = [a[u] + a[u + step] for u in range(step)]
            return _all_reduce(a[0], jnp.add) > 1.0

        tau = lax.cond(
            max_off <= REG_K,
            lambda: _bisect(mx - 1.0, mx, eval_g_reg),
            lambda: lax.cond(
                max_off <= SEG,
                lambda: _bisect(mx - 1.0, mx, eval_g_loop),
                lambda: _bisect(mx - 1.0, mx, eval_g_full),
            ),
        )

        @plsc.parallel_loop(0, NB // 2, unroll=2)
        def _(i):
            base = i * (UNROLL * L)
            for u in range(UNROLL):
                sl = pl.ds(base + u * L, L)
                buf[sl] = jnp.maximum(buf[sl] - tau, 0.0)

        out_issue()

        @plsc.parallel_loop(NB // 2, NB, unroll=2)
        def _(i):
            base = i * (UNROLL * L)
            for u in range(UNROLL):
                sl = pl.ds(base + u * L, L)
                buf[sl] = jnp.maximum(buf[sl] - tau, 0.0)

    H = N // 2
    bufs = (row_a, row_b, row_c)
    in_sems = (sems[0:2], sems[2:4], sems[4:6])
    out_sems = (sems[6:8], sems[8:10], sems[10:12])

    def start_in(j):
        b = j % 3
        return (
            pltpu.async_copy(x_hbm.at[r0 + j, pl.ds(0, H)],
                             bufs[b].at[pl.ds(0, H)], in_sems[b][0]),
            pltpu.async_copy(x_hbm.at[r0 + j, pl.ds(H, H)],
                             bufs[b].at[pl.ds(H, H)], in_sems[b][1]),
        )

    in_cp = [None] * rows_per
    out_cp = [None] * rows_per
    in_cp[0] = start_in(0)
    in_cp[1] = start_in(1)
    for j in range(rows_per):
        b = j % 3
        buf = bufs[b]
        in_cp[j][0].wait()
        outs = []

        def out_issue(j=j, b=b, outs=outs):
            outs.append(pltpu.async_copy(
                bufs[b].at[pl.ds(0, H)],
                out_hbm.at[r0 + j, pl.ds(0, H)], out_sems[b][0]))

        compute_row(buf, in_cp[j][1].wait, out_issue)
        if j + 2 < rows_per:
            if j >= 1:
                for cp in out_cp[j - 1]:
                    cp.wait()
            in_cp[j + 2] = start_in(j + 2)
        outs.append(pltpu.async_copy(
            bufs[b].at[pl.ds(H, H)],
            out_hbm.at[r0 + j, pl.ds(H, H)], out_sems[b][1]))
        out_cp[j] = outs
    for j in range(max(1, rows_per - 3), rows_per):
        for cp in out_cp[j]:
            cp.wait()


def kernel(input):
    return _sparsemax_sc(input)


# speedup vs baseline: 1.1982x; 1.0050x over previous
import functools

import jax
import jax.numpy as jnp
from jax import lax
from jax.experimental import pallas as pl
from jax.experimental.pallas import tpu as pltpu
from jax.experimental.pallas import tpu_sc as plsc

R, N = 128, 32768
L = 16
NV = N // L
SEG = 512
REG_K = 16
UNROLL = 8
NB = NV // UNROLL
N_BISECT = 18
NEG = -1.0e30

_mesh = plsc.VectorSubcoreMesh(core_axis_name="c", subcore_axis_name="s")


def _all_reduce(a, op):
    idx0 = lax.iota(jnp.int32, L)
    for k in (8, 4, 2, 1):
        perm = jnp.bitwise_xor(idx0, k)
        a = op(a, jnp.take_along_axis(a, perm, axis=0))
    return a


def _tree_sum(xs):
    xs = list(xs)
    while len(xs) > 1:
        xs = [xs[i] + xs[i + 1] for i in range(0, len(xs) - 1, 2)] + (
            [xs[-1]] if len(xs) % 2 else []
        )
    return xs[0]


def _bisect(lo, hi, eval_g):
    def body(_, lohi):
        lo, hi = lohi
        tau = 0.5 * (lo + hi)
        big = eval_g(tau)
        return jnp.where(big, tau, lo), jnp.where(big, hi, tau)

    lo, hi = lax.fori_loop(0, N_BISECT, body, (lo, hi))
    return 0.5 * (lo + hi)


@functools.partial(
    pl.kernel,
    mesh=_mesh,
    out_type=jax.ShapeDtypeStruct((R, N), jnp.float32),
    scratch_types=[
        pltpu.VMEM((N,), jnp.float32),
        pltpu.VMEM((N,), jnp.float32),
        pltpu.VMEM((N,), jnp.float32),
        pltpu.VMEM((SEG * L + L,), jnp.float32),
        pltpu.VMEM((NB * L + L,), jnp.int32),
        pltpu.SemaphoreType.DMA,
        pltpu.SemaphoreType.DMA,
        pltpu.SemaphoreType.DMA,
        pltpu.SemaphoreType.DMA,
        pltpu.SemaphoreType.DMA,
        pltpu.SemaphoreType.DMA,
    ],
    compiler_params=pltpu.CompilerParams(needs_layout_passes=False),
)
def _sparsemax_sc(x_hbm, out_hbm, row_a, row_b, row_c, cmp_v, blk_v,
                  si0, si1, si2, so0, so1, so2):
    info = plsc.get_sparse_core_info()
    nc, ns = info.num_cores, info.num_subcores
    nw = nc * ns
    rows_per = R // nw
    wid = lax.axis_index("s") * nc + lax.axis_index("c")
    r0 = wid * rows_per
    lanes = lax.iota(jnp.int32, L)
    dump = SEG * L + lanes
    dump_b = NB * L + lanes

    def compute_row(buf):
        ms0 = tuple(buf[pl.ds(u * L, L)] for u in range(UNROLL))

        @plsc.parallel_loop(1, NB, carry=ms0, unroll=2)
        def ms(i, ms):
            base = i * (UNROLL * L)
            return tuple(
                jnp.maximum(ms[u], buf[pl.ds(base + u * L, L)])
                for u in range(UNROLL)
            )

        step = UNROLL
        while step > 1:
            step //= 2
            ms = tuple(jnp.maximum(ms[u], ms[u + step]) for u in range(step))
        mx = _all_reduce(ms[0], jnp.maximum)

        thr = mx - 1.0

        @plsc.parallel_loop(0, NB, carry=jnp.zeros((L,), jnp.int32), unroll=2)
        def boff(i, boff):
            base = i * (UNROLL * L)
            hots = [buf[pl.ds(base + u * L, L)] > thr for u in range(UNROLL)]
            while len(hots) > 1:
                hots = [hots[k] | hots[k + 1] for k in range(0, len(hots), 2)]
            anyh = hots[0]
            bidx = jnp.where(anyh, boff * L + lanes, dump_b)
            plsc.store_scatter(blk_v, [bidx], jnp.zeros((L,), jnp.int32) + i)
            return boff + anyh.astype(jnp.int32)

        max_boff = _all_reduce(boff, jnp.maximum)[0]

        @plsc.parallel_loop(0, max_boff, carry=jnp.zeros((L,), jnp.int32))
        def off(k, off):
            bid = plsc.load_gather(blk_v, [k * L + lanes])
            bid = jnp.minimum(jnp.maximum(bid, 0), NB - 1)
            valid = k < boff
            for u in range(UNROLL):
                v = plsc.load_gather(buf, [(bid * UNROLL + u) * L + lanes])
                hot = (v > thr) & valid
                slot = jnp.minimum(off, SEG - 1)
                idx = jnp.where(hot, slot * L + lanes, dump)
                plsc.store_scatter(cmp_v, [idx], v)
                off = off + hot.astype(jnp.int32)
            return off

        max_off = _all_reduce(off, jnp.maximum)[0]

        vals = tuple(
            jnp.where(kk < off, cmp_v[pl.ds(kk * L, L)], NEG)
            for kk in range(REG_K)
        )

        def eval_g_reg(tau):
            accs = [jnp.zeros((L,), jnp.float32) for _ in range(4)]
            for kk in range(REG_K):
                accs[kk % 4] = accs[kk % 4] + jnp.maximum(vals[kk] - tau, 0.0)
            return _all_reduce(_tree_sum(accs), jnp.add) > 1.0

        def eval_g_loop(tau):
            @plsc.parallel_loop(0, max_off, carry=jnp.zeros((L,), jnp.float32))
            def a(kk, a):
                v = jnp.where(kk < off, cmp_v[pl.ds(kk * L, L)], NEG)
                return a + jnp.maximum(v - tau, 0.0)

            return _all_reduce(a, jnp.add) > 1.0

        def eval_g_full(tau):
            acc0 = tuple(jnp.zeros((L,), jnp.float32) for _ in range(UNROLL))

            @plsc.parallel_loop(0, NV // UNROLL, carry=acc0, unroll=2)
            def accs(i, accs):
                base = i * (UNROLL * L)
                return tuple(
                    accs[u]
                    + jnp.maximum(buf[pl.ds(base + u * L, L)] - tau, 0.0)
                    for u in range(UNROLL)
                )

            a = list(accs)
            step = UNROLL
            while step > 1:
                step //= 2
                a = [a[u] + a[u + step] for u in range(step)]
            return _all_reduce(a[0], jnp.add) > 1.0

        tau = lax.cond(
            max_off <= REG_K,
            lambda: _bisect(mx - 1.0, mx, eval_g_reg),
            lambda: lax.cond(
                max_off <= SEG,
                lambda: _bisect(mx - 1.0, mx, eval_g_loop),
                lambda: _bisect(mx - 1.0, mx, eval_g_full),
            ),
        )

        @plsc.parallel_loop(0, NV // UNROLL, unroll=2)
        def _(i):
            base = i * (UNROLL * L)
            for u in range(UNROLL):
                sl = pl.ds(base + u * L, L)
                buf[sl] = jnp.maximum(buf[sl] - tau, 0.0)

    bufs = (row_a, row_b, row_c)
    in_sems = (si0, si1, si2)
    out_sems = (so0, so1, so2)
    in_cp = [None] * rows_per
    out_cp = [None] * rows_per
    in_cp[0] = pltpu.async_copy(x_hbm.at[r0], bufs[0], in_sems[0])
    in_cp[1] = pltpu.async_copy(x_hbm.at[r0 + 1], bufs[1], in_sems[1])
    for j in range(rows_per):
        buf = bufs[j % 3]
        in_cp[j].wait()
        compute_row(buf)
        if j + 2 < rows_per:
            if j >= 1:
                out_cp[j - 1].wait()
            in_cp[j + 2] = pltpu.async_copy(
                x_hbm.at[r0 + j + 2], bufs[(j + 2) % 3], in_sems[(j + 2) % 3]
            )
        out_cp[j] = pltpu.async_copy(buf, out_hbm.at[r0 + j], out_sems[j % 3])
    for j in range(max(1, rows_per - 3), rows_per):
        out_cp[j].wait()


def kernel(input):
    return _sparsemax_sc(input)
